# baseline (speedup 1.0000x reference)
import functools

import jax
import jax.numpy as jnp
from jax import lax
from jax.experimental import pallas as pl
from jax.experimental.pallas import tpu as pltpu
from jax.experimental.pallas import tpu_sc as plsc

B = 16384
EMB = 32
CHUNK = 256

_info = plsc.get_sparse_core_info()
_NC, _NS = _info.num_cores, _info.num_subcores
_NW = _NC * _NS
_BPW = B // _NW
_NCHUNK = _BPW // CHUNK


def _sc_gather3(user_table, item_table, idx3):
    mesh = plsc.VectorSubcoreMesh(core_axis_name="c", subcore_axis_name="s")
    out_t = jax.ShapeDtypeStruct((B, EMB), jnp.float32)

    @functools.partial(
        pl.kernel,
        mesh=mesh,
        out_type=[out_t, out_t, out_t],
        scratch_types=[
            pltpu.VMEM((3, _BPW), jnp.int32),
            pltpu.VMEM((CHUNK, EMB), jnp.float32),
            pltpu.VMEM((CHUNK, EMB), jnp.float32),
            pltpu.SemaphoreType.DMA,
            pltpu.SemaphoreType.DMA,
            pltpu.SemaphoreType.DMA,
        ],
    )
    def gather_kernel(ut_hbm, it_hbm, idx_hbm,
                      out_u, out_p, out_n,
                      iv, buf0, buf1, sem0, sem1, semw):
        wid = lax.axis_index("s") * _NC + lax.axis_index("c")
        base = wid * _BPW
        pltpu.sync_copy(idx_hbm.at[wid], iv)
        bufs = (buf0, buf1)
        sems = (sem0, sem1)

        def fire_chunk(tab, t, c, sbuf, sem):
            def blk_body(blk, carry):
                ivv = iv[t, pl.ds(c * CHUNK + blk * 16, 16)]
                for i in range(16):
                    pltpu.make_async_copy(
                        tab.at[pl.ds(ivv[i], 1)],
                        sbuf.at[pl.ds(blk * 16 + i, 1)],
                        sem).start()
                return carry
            lax.fori_loop(0, CHUNK // 16, blk_body, 0)

        def drain_chunk(tab, sbuf, sem):
            def blk_body(blk, carry):
                for i in range(16):
                    pltpu.make_async_copy(
                        tab.at[pl.ds(0, 1)],
                        sbuf.at[pl.ds(blk * 16 + i, 1)],
                        sem).wait()
                return carry
            lax.fori_loop(0, CHUNK // 16, blk_body, 0)

        tabs = (ut_hbm, it_hbm, it_hbm)
        outs = (out_u, out_p, out_n)
        steps = [(t, c) for t in range(3) for c in range(_NCHUNK)]
        fire_chunk(tabs[0], 0, 0, bufs[0], sems[0])
        for s, (t, c) in enumerate(steps):
            sbuf = bufs[s % 2]
            if s + 1 < len(steps):
                tn, cn = steps[s + 1]
                fire_chunk(tabs[tn], tn, cn, bufs[(s + 1) % 2],
                           sems[(s + 1) % 2])
            drain_chunk(tabs[t], sbuf, sems[s % 2])
            copy_out = pltpu.make_async_copy(
                sbuf, outs[t].at[pl.ds(base + c * CHUNK, CHUNK)], semw)
            copy_out.start()
            copy_out.wait()

    return gather_kernel(user_table, item_table, idx3)


def _tc_body(side_ref, w_ref, b_ref, ur_ref, pr_ref, nr_ref, pos_out, neg_out):
    us = jnp.dot(side_ref[...], w_ref[...], preferred_element_type=jnp.float32)
    us = jnp.maximum(us + b_ref[...], 0.0)
    ue = ur_ref[...] + us
    pos_out[...] = jnp.sum(ue * pr_ref[...], axis=1)
    neg_out[...] = jnp.sum(ue * nr_ref[...], axis=1)


def _tc_combine(side, W, b2d, u_rows, p_rows, n_rows):
    score_t = jax.ShapeDtypeStruct((B,), jnp.float32)
    return pl.pallas_call(
        _tc_body,
        out_shape=[score_t, score_t],
    )(side, W, b2d, u_rows, p_rows, n_rows)


def kernel(u, pos, neg, side, user_table, item_table, W, b):
    ui = u.reshape(-1).astype(jnp.int32)
    pi = pos.reshape(-1).astype(jnp.int32)
    ni = neg.reshape(-1).astype(jnp.int32)
    idx3 = jnp.stack([ui, pi, ni]).reshape(3, _NW, _BPW).transpose(1, 0, 2)

    u_rows, p_rows, n_rows = _sc_gather3(
        jax.new_ref(user_table), jax.new_ref(item_table), idx3)
    pos_s, neg_s = _tc_combine(side, W, b.reshape(1, EMB),
                               u_rows, p_rows, n_rows)
    return (pos_s, neg_s)
